# baseline (speedup 1.0000x reference)
import functools

import jax
import jax.numpy as jnp
from jax import lax
from jax.experimental import pallas as pl
from jax.experimental.pallas import tpu as pltpu
from jax.experimental.pallas import tpu_sc as plsc


def _sc_gather(emb, idx, chunk=128):
    n = idx.shape[0]
    d = emb.shape[1]
    info = plsc.get_sparse_core_info()
    nw = info.num_cores * info.num_subcores
    per_w = n // nw
    assert per_w * nw == n and per_w % chunk == 0
    n_chunks = per_w // chunk

    assert n_chunks >= 12 and n_chunks % 4 == 0
    mesh = plsc.VectorSubcoreMesh(core_axis_name="c", subcore_axis_name="s")
    nbuf = 4

    @functools.partial(
        pl.kernel,
        mesh=mesh,
        out_type=jax.ShapeDtypeStruct((n, d), emb.dtype),
        scratch_types=(
            [pltpu.VMEM((chunk,), jnp.int32)] * nbuf
            + [pltpu.VMEM((chunk, d), emb.dtype)] * nbuf
            + [pltpu.SemaphoreType.DMA] * (3 * nbuf)
        ),
    )
    def gather_kernel(emb_hbm, idx_hbm, out_hbm, *scr):
        ivs = scr[:nbuf]
        rvs = scr[nbuf:2 * nbuf]
        sis = scr[2 * nbuf:3 * nbuf]
        sgs = scr[3 * nbuf:4 * nbuf]
        sws = scr[4 * nbuf:5 * nbuf]
        wid = lax.axis_index("s") * info.num_cores + lax.axis_index("c")
        base = wid * per_w

        def start_idx(c, b):
            pltpu.async_copy(idx_hbm.at[pl.ds(base + c * chunk, chunk)],
                             ivs[b], sis[b])

        def wait_idx(b):
            pltpu.make_async_copy(idx_hbm.at[pl.ds(base, chunk)],
                                  ivs[b], sis[b]).wait()

        def start_gather(b):
            pltpu.async_copy(emb_hbm.at[ivs[b]], rvs[b], sgs[b])

        def wait_gather(b):
            pltpu.make_async_copy(emb_hbm.at[ivs[b]], rvs[b], sgs[b]).wait()

        def start_wb(c, b):
            pltpu.async_copy(rvs[b],
                             out_hbm.at[pl.ds(base + c * chunk, chunk)],
                             sws[b])

        def wait_wb(b):
            pltpu.make_async_copy(rvs[b],
                                  out_hbm.at[pl.ds(base, chunk)],
                                  sws[b]).wait()

        for c in range(nbuf):
            start_idx(c, c)
        wait_idx(0)
        start_gather(0)
        for c in range(1, nbuf):
            b, bm1 = c % nbuf, (c - 1) % nbuf
            wait_idx(b)
            start_gather(b)
            wait_gather(bm1)
            start_wb(c - 1, bm1)
            start_idx(c + 3, bm1)

        def group(g, carry):
            c0 = g * nbuf
            for b in range(nbuf):
                c = c0 + b
                bm1 = (b - 1) % nbuf
                wait_idx(b)
                wait_wb(b)
                start_gather(b)
                wait_gather(bm1)
                start_wb(c - 1, bm1)
                start_idx(c + 3, bm1)
            return carry

        lax.fori_loop(1, n_chunks // nbuf - 1, group, 0)

        for c in range(n_chunks - nbuf, n_chunks):
            b, bm1 = c % nbuf, (c - 1) % nbuf
            wait_idx(b)
            wait_wb(b)
            start_gather(b)
            wait_gather(bm1)
            start_wb(c - 1, bm1)
            if c + 3 < n_chunks:
                start_idx(c + 3, bm1)
        last = (n_chunks - 1) % nbuf
        wait_gather(last)
        start_wb(n_chunks - 1, last)
        for b in range(nbuf):
            wait_wb(b)

    return gather_kernel(emb, idx)


def _rnn_chunk_step(x_ref, h1i_ref, h2i_ref, wih0_ref, whh0_ref, bih0_ref,
                    bhh0_ref, wih1_ref, whh1_ref, bih1_ref, bhh1_ref,
                    fc1w_ref, fc1b_ref, fc2w_ref, fc2b_ref,
                    *out_refs, n_steps, final):
    t = pl.program_id(0)
    if final:
        out_ref, h1_ref, h2_ref = out_refs[0], out_refs[1], out_refs[2]
    else:
        h1_ref, h2_ref = out_refs[0], out_refs[1]

    @pl.when(t == 0)
    def _():
        h1_ref[...] = h1i_ref[...]
        h2_ref[...] = h2i_ref[...]

    steps_per_body = x_ref.shape[0]
    h1 = h1_ref[...]
    h2 = h2_ref[...]
    for s in range(steps_per_body):
        x_t = x_ref[s].astype(jnp.float32)
        a1 = (jnp.dot(x_t, wih0_ref[...], preferred_element_type=jnp.float32)
              + bih0_ref[...]
              + jnp.dot(h1, whh0_ref[...], preferred_element_type=jnp.float32)
              + bhh0_ref[...])
        h1 = jnp.tanh(a1)
        a2 = (jnp.dot(h1, wih1_ref[...], preferred_element_type=jnp.float32)
              + bih1_ref[...]
              + jnp.dot(h2, whh1_ref[...], preferred_element_type=jnp.float32)
              + bhh1_ref[...])
        h2 = jnp.tanh(a2)
    h1_ref[...] = h1
    h2_ref[...] = h2

    if final:
        @pl.when(t == n_steps - 1)
        def _():
            r = jnp.maximum(
                jnp.dot(h2, fc1w_ref[...], preferred_element_type=jnp.float32)
                + fc1b_ref[...], 0.0)
            out_ref[...] = (jnp.dot(r, fc2w_ref[...],
                                    preferred_element_type=jnp.float32)
                            + fc2b_ref[...])


def _tc_rnn_chunk(xemb, h1_in, h2_in, weights, final, spb=4):
    l, b, e = xemb.shape
    hid = h1_in.shape[1]
    f1 = weights[8].shape[1]
    assert l % spb == 0
    n_body = l // spb

    full = lambda shape: pl.BlockSpec(shape, lambda t: (0,) * len(shape))
    hspec = full((b, hid))
    out_specs = [hspec, hspec]
    out_shape = [jax.ShapeDtypeStruct((b, hid), jnp.float32),
                 jax.ShapeDtypeStruct((b, hid), jnp.float32)]
    if final:
        out_specs = [full((b, 1))] + out_specs
        out_shape = [jax.ShapeDtypeStruct((b, 1), jnp.float32)] + out_shape
    return pl.pallas_call(
        functools.partial(_rnn_chunk_step, n_steps=n_body, final=final),
        grid=(n_body,),
        in_specs=[
            pl.BlockSpec((spb, b, e), lambda t: (t, 0, 0)),
            hspec, hspec,
            full((e, hid)), full((hid, hid)), full((1, hid)), full((1, hid)),
            full((hid, hid)), full((hid, hid)), full((1, hid)), full((1, hid)),
            full((hid, f1)), full((1, f1)), full((f1, 1)), full((1, 1)),
        ],
        out_specs=out_specs,
        out_shape=out_shape,
        compiler_params=pltpu.CompilerParams(
            dimension_semantics=("arbitrary",)),
    )(xemb, h1_in, h2_in, *weights)


_N_CHUNKS = 5


def kernel(x, emb, W_ih0, W_hh0, b_ih0, b_hh0, W_ih1, W_hh1, b_ih1, b_hh1,
           fc1_w, fc1_b, fc2_w, fc2_b):
    b, l = x.shape
    e = emb.shape[1]
    hid = W_hh0.shape[0]
    idx = x.astype(jnp.int32).T.reshape(-1)
    sizes = [24, 28, 36, 48, 64] if l == 200 else [l // _N_CHUNKS] * _N_CHUNKS
    starts = [sum(sizes[:i]) for i in range(len(sizes))]
    gs = [_sc_gather(emb, lax.slice_in_dim(idx, s * b, (s + lc) * b))
          .reshape(lc, b, e)
          for s, lc in zip(starts, sizes)]
    weights = (W_ih0.T, W_hh0.T, b_ih0.reshape(1, -1), b_hh0.reshape(1, -1),
               W_ih1.T, W_hh1.T, b_ih1.reshape(1, -1), b_hh1.reshape(1, -1),
               fc1_w.T, fc1_b.reshape(1, -1), fc2_w.T, fc2_b.reshape(1, -1))
    h1 = jnp.zeros((b, hid), jnp.float32)
    h2 = jnp.zeros((b, hid), jnp.float32)
    for c in range(len(gs) - 1):
        h1, h2 = _tc_rnn_chunk(gs[c], h1, h2, weights, final=False)
    out, _, _ = _tc_rnn_chunk(gs[-1], h1, h2, weights, final=True)
    return out


# speedup vs baseline: 1.0464x; 1.0464x over previous
import functools

import jax
import jax.numpy as jnp
from jax import lax
from jax.experimental import pallas as pl
from jax.experimental.pallas import tpu as pltpu
from jax.experimental.pallas import tpu_sc as plsc


def _sc_gather(emb, idx, chunk=128):
    n = idx.shape[0]
    d = emb.shape[1]
    info = plsc.get_sparse_core_info()
    nw = info.num_cores * info.num_subcores
    per_w = n // nw
    assert per_w * nw == n and per_w % chunk == 0
    n_chunks = per_w // chunk

    assert n_chunks >= 12 and n_chunks % 4 == 0
    mesh = plsc.VectorSubcoreMesh(core_axis_name="c", subcore_axis_name="s")
    nbuf = 4

    @functools.partial(
        pl.kernel,
        mesh=mesh,
        out_type=jax.ShapeDtypeStruct((n, d), emb.dtype),
        scratch_types=(
            [pltpu.VMEM((chunk,), jnp.int32)] * nbuf
            + [pltpu.VMEM((chunk, d), emb.dtype)] * nbuf
            + [pltpu.SemaphoreType.DMA] * (3 * nbuf)
        ),
    )
    def gather_kernel(emb_hbm, idx_hbm, out_hbm, *scr):
        ivs = scr[:nbuf]
        rvs = scr[nbuf:2 * nbuf]
        sis = scr[2 * nbuf:3 * nbuf]
        sgs = scr[3 * nbuf:4 * nbuf]
        sws = scr[4 * nbuf:5 * nbuf]
        wid = lax.axis_index("s") * info.num_cores + lax.axis_index("c")
        base = wid * per_w

        def start_idx(c, b):
            pltpu.async_copy(idx_hbm.at[pl.ds(base + c * chunk, chunk)],
                             ivs[b], sis[b])

        def wait_idx(b):
            pltpu.make_async_copy(idx_hbm.at[pl.ds(base, chunk)],
                                  ivs[b], sis[b]).wait()

        def start_gather(b):
            pltpu.async_copy(emb_hbm.at[ivs[b]], rvs[b], sgs[b])

        def wait_gather(b):
            pltpu.make_async_copy(emb_hbm.at[ivs[b]], rvs[b], sgs[b]).wait()

        def start_wb(c, b):
            pltpu.async_copy(rvs[b],
                             out_hbm.at[pl.ds(base + c * chunk, chunk)],
                             sws[b])

        def wait_wb(b):
            pltpu.make_async_copy(rvs[b],
                                  out_hbm.at[pl.ds(base, chunk)],
                                  sws[b]).wait()

        for c in range(nbuf):
            start_idx(c, c)
        wait_idx(0)
        start_gather(0)
        for c in range(1, nbuf):
            b, bm1 = c % nbuf, (c - 1) % nbuf
            wait_idx(b)
            start_gather(b)
            wait_gather(bm1)
            start_wb(c - 1, bm1)
            start_idx(c + 3, bm1)

        def group(g, carry):
            c0 = g * nbuf
            for b in range(nbuf):
                c = c0 + b
                bm1 = (b - 1) % nbuf
                wait_idx(b)
                wait_wb(b)
                start_gather(b)
                wait_gather(bm1)
                start_wb(c - 1, bm1)
                start_idx(c + 3, bm1)
            return carry

        lax.fori_loop(1, n_chunks // nbuf - 1, group, 0)

        for c in range(n_chunks - nbuf, n_chunks):
            b, bm1 = c % nbuf, (c - 1) % nbuf
            wait_idx(b)
            wait_wb(b)
            start_gather(b)
            wait_gather(bm1)
            start_wb(c - 1, bm1)
            if c + 3 < n_chunks:
                start_idx(c + 3, bm1)
        last = (n_chunks - 1) % nbuf
        wait_gather(last)
        start_wb(n_chunks - 1, last)
        for b in range(nbuf):
            wait_wb(b)

    return gather_kernel(emb, idx)


def _rnn_chunk_step(x_ref, h1i_ref, h2i_ref, wih0_ref, whh0_ref, bih0_ref,
                    bhh0_ref, wih1_ref, whh1_ref, bih1_ref, bhh1_ref,
                    fc1w_ref, fc1b_ref, fc2w_ref, fc2b_ref,
                    *out_refs, n_steps, final):
    t = pl.program_id(0)
    if final:
        out_ref, h1_ref, h2_ref = out_refs[0], out_refs[1], out_refs[2]
    else:
        h1_ref, h2_ref = out_refs[0], out_refs[1]

    @pl.when(t == 0)
    def _():
        h1_ref[...] = h1i_ref[...]
        h2_ref[...] = h2i_ref[...]

    steps_per_body = x_ref.shape[0]
    h1 = h1_ref[...]
    h2 = h2_ref[...]
    for s in range(steps_per_body):
        x_t = x_ref[s].astype(jnp.float32)
        a1 = (jnp.dot(x_t, wih0_ref[...], preferred_element_type=jnp.float32)
              + bih0_ref[...]
              + jnp.dot(h1, whh0_ref[...], preferred_element_type=jnp.float32)
              + bhh0_ref[...])
        h1 = jnp.tanh(a1)
        a2 = (jnp.dot(h1, wih1_ref[...], preferred_element_type=jnp.float32)
              + bih1_ref[...]
              + jnp.dot(h2, whh1_ref[...], preferred_element_type=jnp.float32)
              + bhh1_ref[...])
        h2 = jnp.tanh(a2)
    h1_ref[...] = h1
    h2_ref[...] = h2

    if final:
        @pl.when(t == n_steps - 1)
        def _():
            r = jnp.maximum(
                jnp.dot(h2, fc1w_ref[...], preferred_element_type=jnp.float32)
                + fc1b_ref[...], 0.0)
            out_ref[...] = (jnp.dot(r, fc2w_ref[...],
                                    preferred_element_type=jnp.float32)
                            + fc2b_ref[...])


def _tc_rnn_chunk(xemb, h1_in, h2_in, weights, final, spb=5):
    l, b, e = xemb.shape
    hid = h1_in.shape[1]
    f1 = weights[8].shape[1]
    assert l % spb == 0
    n_body = l // spb

    full = lambda shape: pl.BlockSpec(shape, lambda t: (0,) * len(shape))
    hspec = full((b, hid))
    out_specs = [hspec, hspec]
    out_shape = [jax.ShapeDtypeStruct((b, hid), jnp.float32),
                 jax.ShapeDtypeStruct((b, hid), jnp.float32)]
    if final:
        out_specs = [full((b, 1))] + out_specs
        out_shape = [jax.ShapeDtypeStruct((b, 1), jnp.float32)] + out_shape
    return pl.pallas_call(
        functools.partial(_rnn_chunk_step, n_steps=n_body, final=final),
        grid=(n_body,),
        in_specs=[
            pl.BlockSpec((spb, b, e), lambda t: (t, 0, 0)),
            hspec, hspec,
            full((e, hid)), full((hid, hid)), full((1, hid)), full((1, hid)),
            full((hid, hid)), full((hid, hid)), full((1, hid)), full((1, hid)),
            full((hid, f1)), full((1, f1)), full((f1, 1)), full((1, 1)),
        ],
        out_specs=out_specs,
        out_shape=out_shape,
        compiler_params=pltpu.CompilerParams(
            dimension_semantics=("arbitrary",)),
    )(xemb, h1_in, h2_in, *weights)


_N_CHUNKS = 5


def kernel(x, emb, W_ih0, W_hh0, b_ih0, b_hh0, W_ih1, W_hh1, b_ih1, b_hh1,
           fc1_w, fc1_b, fc2_w, fc2_b):
    b, l = x.shape
    e = emb.shape[1]
    hid = W_hh0.shape[0]
    idx = x.astype(jnp.int32).T.reshape(-1)
    sizes = [20, 60, 60, 60] if l == 200 else [l // _N_CHUNKS] * _N_CHUNKS
    starts = [sum(sizes[:i]) for i in range(len(sizes))]
    gs = [_sc_gather(emb, lax.slice_in_dim(idx, s * b, (s + lc) * b))
          .reshape(lc, b, e)
          for s, lc in zip(starts, sizes)]
    weights = (W_ih0.T, W_hh0.T, b_ih0.reshape(1, -1), b_hh0.reshape(1, -1),
               W_ih1.T, W_hh1.T, b_ih1.reshape(1, -1), b_hh1.reshape(1, -1),
               fc1_w.T, fc1_b.reshape(1, -1), fc2_w.T, fc2_b.reshape(1, -1))
    h1 = jnp.zeros((b, hid), jnp.float32)
    h2 = jnp.zeros((b, hid), jnp.float32)
    for c in range(len(gs) - 1):
        h1, h2 = _tc_rnn_chunk(gs[c], h1, h2, weights, final=False)
    out, _, _ = _tc_rnn_chunk(gs[-1], h1, h2, weights, final=True)
    return out
